# baseline (speedup 1.0000x reference)
import functools

import jax
import jax.numpy as jnp
from jax import lax
from jax.experimental import pallas as pl
from jax.experimental.pallas import tpu as pltpu
from jax.experimental.pallas import tpu_sc as plsc

_MAX_SEQ_LEN = 8192
_DIM = 1024
_N = 32768

_NC = 2
_NS = 16
_NW = _NC * _NS
_B_PER_W = _N // _NW
_CHUNK = 32
_N_CHUNKS = _B_PER_W // _CHUNK


def _make_gather():
    mesh = plsc.VectorSubcoreMesh(core_axis_name="c", subcore_axis_name="s")

    @functools.partial(
        pl.kernel,
        mesh=mesh,
        out_type=jax.ShapeDtypeStruct((_N, _DIM), jnp.float32),
        scratch_types=[
            pltpu.SMEM((_B_PER_W,), jnp.int32),
            pltpu.VMEM_SHARED((_NS, _B_PER_W), jnp.int32),
            pltpu.SemaphoreType.DMA,
        ],
    )
    def gather(table_hbm, idx_hbm, out_hbm, idx_s, idx_sp, sem):
        cid = lax.axis_index("c")
        sid = lax.axis_index("s")
        wid = sid * _NC + cid
        base = wid * _B_PER_W
        pltpu.sync_copy(idx_hbm.at[wid], idx_sp.at[sid])
        pltpu.sync_copy(idx_sp.at[sid], idx_s)

        def issue_chunk(j):
            def row(k, carry):
                p = j * _CHUNK + k
                i = idx_s[p]
                pltpu.async_copy(
                    table_hbm.at[pl.ds(i, 1)],
                    out_hbm.at[pl.ds(base + p, 1)], sem)
                return carry
            lax.fori_loop(0, _CHUNK, row, 0)

        def drain_chunk(j):
            def row(k, carry):
                pltpu.make_async_copy(
                    table_hbm.at[pl.ds(0, 1)],
                    out_hbm.at[pl.ds(base, 1)], sem).wait()
                return carry
            lax.fori_loop(0, _CHUNK, row, 0)

        issue_chunk(0)
        issue_chunk(1)

        def body(t, carry):
            drain_chunk(t)
            issue_chunk(t + 2)
            return carry

        lax.fori_loop(0, _N_CHUNKS - 2, body, 0)
        drain_chunk(_N_CHUNKS - 2)
        drain_chunk(_N_CHUNKS - 1)

    return gather


_gather = _make_gather()


def kernel(seq_len_or_indices, embedding):
    idx = seq_len_or_indices.astype(jnp.int32).reshape(_NW, _B_PER_W)
    table = embedding.reshape(_MAX_SEQ_LEN, _DIM)
    return _gather(table, idx)


# speedup vs baseline: 37.2947x; 37.2947x over previous
import functools

import jax
import jax.numpy as jnp
from jax import lax
from jax.experimental import pallas as pl
from jax.experimental.pallas import tpu as pltpu
from jax.experimental.pallas import tpu_sc as plsc

_MAX_SEQ_LEN = 8192
_DIM = 1024
_N = 32768

_NC = 2
_NS = 16
_NW = _NC * _NS
_B_PER_W = _N // _NW
_CHUNK = 16
_N_CHUNKS = _B_PER_W // _CHUNK
_NBUF = 6
_LOOK = 3


def _make_gather():
    mesh = plsc.VectorSubcoreMesh(core_axis_name="c", subcore_axis_name="s")

    @functools.partial(
        pl.kernel,
        mesh=mesh,
        out_type=jax.ShapeDtypeStruct((_N, _DIM), jnp.float32),
        scratch_types=[
            pltpu.SMEM((_B_PER_W,), jnp.int32),
            pltpu.VMEM_SHARED((_NS, _B_PER_W), jnp.int32),
            pltpu.VMEM_SHARED((_NBUF, _NS * _CHUNK, _DIM), jnp.float32),
            pltpu.SemaphoreType.DMA,
            pltpu.SemaphoreType.DMA,
            pltpu.SemaphoreType.DMA,
            pltpu.SemaphoreType.DMA,
            pltpu.SemaphoreType.DMA,
            pltpu.SemaphoreType.DMA,
            pltpu.SemaphoreType.DMA,
            pltpu.SemaphoreType.DMA,
            pltpu.SemaphoreType.DMA,
            pltpu.SemaphoreType.DMA,
            pltpu.SemaphoreType.DMA,
            pltpu.SemaphoreType.DMA,
        ],
    )
    def gather(table_hbm, idx_hbm, out_hbm, idx_s, idx_sp, sp,
               gsem0, gsem1, gsem2, gsem3, gsem4, gsem5,
               osem0, osem1, osem2, osem3, osem4, osem5):
        gsems = (gsem0, gsem1, gsem2, gsem3, gsem4, gsem5)
        osems = (osem0, osem1, osem2, osem3, osem4, osem5)
        cid = lax.axis_index("c")
        sid = lax.axis_index("s")
        wid = sid * _NC + cid
        base = wid * _B_PER_W
        slot = sid * _CHUNK
        pltpu.sync_copy(idx_hbm.at[wid], idx_sp.at[sid])
        pltpu.sync_copy(idx_sp.at[sid], idx_s)

        def g_start(j, b):
            def row(k, carry):
                i = idx_s[j * _CHUNK + k]
                pltpu.async_copy(
                    table_hbm.at[pl.ds(i, 1)],
                    sp.at[b, pl.ds(slot + k, 1)], gsems[b])
                return carry
            lax.fori_loop(0, _CHUNK, row, 0)

        def g_wait(j, b):
            def row(k, carry):
                pltpu.make_async_copy(
                    table_hbm.at[pl.ds(0, 1)],
                    sp.at[b, pl.ds(slot + k, 1)], gsems[b]).wait()
                return carry
            lax.fori_loop(0, _CHUNK, row, 0)

        def o_start(j, b):
            pltpu.async_copy(
                sp.at[b, pl.ds(slot, _CHUNK)],
                out_hbm.at[pl.ds(base + j * _CHUNK, _CHUNK)], osems[b])

        def o_wait(j, b):
            pltpu.make_async_copy(
                sp.at[b, pl.ds(slot, _CHUNK)],
                out_hbm.at[pl.ds(base + j * _CHUNK, _CHUNK)], osems[b]).wait()

        for s in range(_LOOK):
            g_start(s, s % _NBUF)
        for s in range(_LOOK):
            b = s % _NBUF
            g_wait(s, b)
            o_start(s, b)
            g_start(s + _LOOK, (s + _LOOK) % _NBUF)

        first_steady = _LOOK
        last_steady = _N_CHUNKS - 1 - _LOOK
        n_steady = last_steady - first_steady + 1
        n_unrolled = (n_steady // _NBUF) * _NBUF

        def body(u, carry):
            for v in range(_NBUF):
                s = first_steady + _NBUF * u + v
                b = (first_steady + v) % _NBUF
                g_wait(s, b)
                o_start(s, b)
                o_wait(s - _LOOK, (first_steady + v - _LOOK) % _NBUF)
                g_start(s + _LOOK, (first_steady + v + _LOOK) % _NBUF)
            return carry

        lax.fori_loop(0, n_unrolled // _NBUF, body, 0)
        for s in range(first_steady + n_unrolled, _N_CHUNKS):
            b = s % _NBUF
            g_wait(s, b)
            o_start(s, b)
            o_wait(s - _LOOK, (s - _LOOK) % _NBUF)
            if s + _LOOK < _N_CHUNKS:
                g_start(s + _LOOK, (s + _LOOK) % _NBUF)
        for s in range(_N_CHUNKS - _LOOK, _N_CHUNKS):
            o_wait(s, s % _NBUF)

    return gather


_gather = _make_gather()


def kernel(seq_len_or_indices, embedding):
    idx = seq_len_or_indices.astype(jnp.int32).reshape(_NW, _B_PER_W)
    table = embedding.reshape(_MAX_SEQ_LEN, _DIM)
    return _gather(table, idx)
